# baseline (speedup 1.0000x reference)
import jax
import jax.numpy as jnp
from jax.experimental import pallas as pl
from jax.experimental.pallas import tpu as pltpu

B, R, C = 4, 8192, 1024
_ROW_BLK_NORM = 1024
_ROW_BLK_MUL = 512


def _sumsq_tree(x):
    sq = x * x
    c = [sq[..., 128 * i:128 * (i + 1)] for i in range(8)]
    t = [c[0] + c[1], c[2] + c[3], c[4] + c[5], c[6] + c[7]]
    u = [t[0] + t[1], t[2] + t[3]]
    acc128 = u[0] + u[1]
    tr = jnp.swapaxes(acc128, 1, 2)
    acc = tr[:, 0:8, :]
    for g in range(1, 16):
        acc = acc + tr[:, 8 * g:8 * (g + 1), :]
    a4 = acc[:, 0:4, :] + acc[:, 4:8, :]
    a2 = a4[:, 0:2, :] + a4[:, 2:4, :]
    return a2[:, 0, :] + a2[:, 1, :]


def _norms_body(w_ref, mag_ref):
    s = _sumsq_tree(w_ref[...])
    mag_ref[...] = jnp.where(s == 0.0, 0.0, s * jax.lax.rsqrt(s))


def _mul_body(w_ref, mag_ref, thr_ref, out_ref):
    thr = thr_ref[...][:, 0:1]
    mask = (mag_ref[...] >= thr).astype(jnp.float32)
    out_ref[...] = w_ref[...] * mask[:, :, None]


def _sc_select(mags, karr):
    from jax.experimental.pallas import tpu_sc as plsc

    import dataclasses

    mesh = plsc.VectorSubcoreMesh(core_axis_name="c", subcore_axis_name="s")
    cp = pltpu.CompilerParams()
    if "needs_layout_passes" in pltpu.CompilerParams.__dataclass_fields__:
        cp = dataclasses.replace(cp, needs_layout_passes=False)

    @pl.kernel(out_type=jax.ShapeDtypeStruct((1, 16), jnp.int32),
               mesh=mesh,
               compiler_params=cp,
               scratch_types=[pltpu.VMEM((R,), jnp.float32),
                              pltpu.VMEM((16,), jnp.int32),
                              pltpu.VMEM((16,), jnp.int32),
                              pltpu.SemaphoreType.DMA])
    def sel(mags_hbm, k_hbm, out_hbm, buf, kbuf, obuf, sem):
        core = jax.lax.axis_index("c")
        sub = jax.lax.axis_index("s")

        @pl.when(jnp.logical_and(core == 0, sub == 0))
        def _():
            pltpu.async_copy(mags_hbm.at[0], buf, sem).wait()
            pltpu.async_copy(k_hbm.at[0], kbuf, sem).wait()
            k = kbuf[...][0]

            def bit_step(i, prefix):
                trial = prefix | (1 << (30 - i))

                def slice_step(j, cnt):
                    v = buf[pl.ds(j * 16, 16)]
                    vb = jax.lax.bitcast_convert_type(v, jnp.int32)
                    return cnt + jnp.where(vb >= trial, 1, 0)

                cnt16 = jax.lax.fori_loop(0, R // 16, slice_step,
                                          jnp.zeros((16,), jnp.int32))
                total = jnp.sum(cnt16)
                return jnp.where(total >= k, trial, prefix)

            prefix = jax.lax.fori_loop(0, 31, bit_step, 0)
            obuf[...] = jnp.full((16,), prefix, jnp.int32)
            pltpu.async_copy(obuf, out_hbm.at[0], sem).wait()

    return sel(mags, karr)


@jax.jit
def kernel(weight_params, logits, W, b):
    keep_fraction = jax.nn.sigmoid(logits @ W + b)
    items_to_keep = jnp.clip((keep_fraction * R).astype(jnp.int32), 1, None)
    karr = jnp.broadcast_to(items_to_keep, (B, 16))

    mags_list = []
    thr_list = []
    for bb in range(B):
        mags_b = pl.pallas_call(
            _norms_body,
            grid=(R // _ROW_BLK_NORM,),
            in_specs=[pl.BlockSpec((1, _ROW_BLK_NORM, C),
                                   lambda r, bb=bb: (bb, r, 0))],
            out_specs=pl.BlockSpec((1, _ROW_BLK_NORM), lambda r: (0, r)),
            out_shape=jax.ShapeDtypeStruct((1, R), jnp.float32),
        )(weight_params)
        mags_list.append(mags_b)
        thr_list.append(_sc_select(mags_b, karr[bb:bb + 1]))
    mags = jnp.concatenate(mags_list, axis=0)
    thr_bits = jnp.concatenate(thr_list, axis=0)
    thr = jax.lax.bitcast_convert_type(thr_bits, jnp.float32)

    out = pl.pallas_call(
        _mul_body,
        grid=(R // _ROW_BLK_MUL,),
        in_specs=[
            pl.BlockSpec((B, _ROW_BLK_MUL, C), lambda r: (0, r, 0)),
            pl.BlockSpec((B, _ROW_BLK_MUL), lambda r: (0, r)),
            pl.BlockSpec((B, 16), lambda r: (0, 0)),
        ],
        out_specs=pl.BlockSpec((B, _ROW_BLK_MUL, C), lambda r: (0, r, 0)),
        out_shape=jax.ShapeDtypeStruct((B, R, C), jnp.float32),
    )(weight_params, mags, thr)

    return out


# speedup vs baseline: 2.6062x; 2.6062x over previous
import jax
import jax.numpy as jnp
from jax.experimental import pallas as pl
from jax.experimental.pallas import tpu as pltpu

B, R, C = 4, 8192, 1024
_ROW_BLK_NORM = 1024
_ROW_BLK_MUL = 512


def _sumsq_tree(x):
    sq = x * x
    c = [sq[..., 128 * i:128 * (i + 1)] for i in range(8)]
    t = [c[0] + c[1], c[2] + c[3], c[4] + c[5], c[6] + c[7]]
    u = [t[0] + t[1], t[2] + t[3]]
    acc128 = u[0] + u[1]
    tr = jnp.swapaxes(acc128, 1, 2)
    acc = tr[:, 0:8, :]
    for g in range(1, 16):
        acc = acc + tr[:, 8 * g:8 * (g + 1), :]
    a4 = acc[:, 0:4, :] + acc[:, 4:8, :]
    a2 = a4[:, 0:2, :] + a4[:, 2:4, :]
    return a2[:, 0, :] + a2[:, 1, :]


def _norms_body(w_ref, mag_ref):
    s = _sumsq_tree(w_ref[...])
    mag_ref[...] = jnp.where(s == 0.0, 0.0, s * jax.lax.rsqrt(s))


def _select_body(mag_ref, k_ref, mask_ref):
    k = jnp.stack([k_ref[0, i] for i in range(B)]).reshape(B, 1)

    bits = pltpu.bitcast(mag_ref[...], jnp.int32)

    def step(i, prefix):
        trial = prefix | (1 << (30 - i))
        cnt = jnp.sum((bits >= trial).astype(jnp.int32), axis=-1,
                      keepdims=True)
        return jnp.where(cnt >= k, trial, prefix)

    prefix = jax.lax.fori_loop(0, 31, step, jnp.zeros((B, 1), jnp.int32))
    mask_ref[...] = (bits >= prefix).astype(jnp.float32)


def _mul_body(w_ref, mask_ref, out_ref):
    out_ref[...] = w_ref[...] * mask_ref[...][:, :, None]


@jax.jit
def kernel(weight_params, logits, W, b):
    keep_fraction = jax.nn.sigmoid(logits @ W + b)
    items_to_keep = jnp.clip((keep_fraction * R).astype(jnp.int32), 1, None)
    k = jnp.squeeze(items_to_keep, axis=-1).reshape(1, B)

    mags = pl.pallas_call(
        _norms_body,
        grid=(R // _ROW_BLK_NORM,),
        in_specs=[pl.BlockSpec((B, _ROW_BLK_NORM, C), lambda r: (0, r, 0))],
        out_specs=pl.BlockSpec((B, _ROW_BLK_NORM), lambda r: (0, r)),
        out_shape=jax.ShapeDtypeStruct((B, R), jnp.float32),
    )(weight_params)

    mask = pl.pallas_call(
        _select_body,
        in_specs=[
            pl.BlockSpec((B, R), lambda: (0, 0)),
            pl.BlockSpec(memory_space=pltpu.SMEM),
        ],
        out_specs=pl.BlockSpec((B, R), lambda: (0, 0)),
        out_shape=jax.ShapeDtypeStruct((B, R), jnp.float32),
    )(mags, k)

    out = pl.pallas_call(
        _mul_body,
        grid=(R // _ROW_BLK_MUL,),
        in_specs=[
            pl.BlockSpec((B, _ROW_BLK_MUL, C), lambda r: (0, r, 0)),
            pl.BlockSpec((B, _ROW_BLK_MUL), lambda r: (0, r)),
        ],
        out_specs=pl.BlockSpec((B, _ROW_BLK_MUL, C), lambda r: (0, r, 0)),
        out_shape=jax.ShapeDtypeStruct((B, R, C), jnp.float32),
    )(weight_params, mask)

    return out
